# baseline (speedup 1.0000x reference)
import functools

import jax
import jax.numpy as jnp
from jax import lax
from jax.experimental import pallas as pl
from jax.experimental.pallas import tpu as pltpu
from jax.experimental.pallas import tpu_sc as plsc

NUM_CODES = 8192
DIM = 64
ROWS = 16384
R_BLK = 128
C_BLK = 1024
LANES = 128
SUB = 8
GRPS = C_BLK // LANES
R3 = R_BLK // SUB

SC_CORES = 2
SC_SUBCORES = 16
SC_WORKERS = SC_CORES * SC_SUBCORES
ROWS_PER_WORKER = ROWS // SC_WORKERS


def _argmin_body(x_ref, e_ref, et_ref, idx_ref, loss_ref, e2_ref, acc_ref):
    i = pl.program_id(0)

    @pl.when(i == 0)
    def _():
        acc_ref[0] = 0.0

        def pre(j, c):
            e = e_ref[pl.ds(j * C_BLK, C_BLK), :]
            e2 = jnp.sum(e * e, axis=1)
            e2_ref[:, pl.ds(j * C_BLK, C_BLK)] = jnp.broadcast_to(
                e2[None, :], (SUB, C_BLK))
            return c

        lax.fori_loop(0, NUM_CODES // C_BLK, pre, 0)

    x = x_ref[...]
    xm2 = -2.0 * x
    x2 = jnp.sum(x * x, axis=1, keepdims=True)
    x2b = jnp.broadcast_to(x2, (R_BLK, LANES)).reshape(R3, SUB, LANES)

    def chunk(j, carry):
        run_min, run_grp = carry
        et = et_ref[:, pl.ds(j * C_BLK, C_BLK)]
        m2 = lax.dot_general(
            xm2, et, (((1,), (0,)), ((), ())),
            preferred_element_type=jnp.float32)
        m3 = m2.reshape(R3, SUB, C_BLK)
        e2c = e2_ref[:, pl.ds(j * C_BLK, C_BLK)]
        for g in range(GRPS):
            e2g = e2c[None, :, g * LANES:(g + 1) * LANES]
            mg = m3[:, :, g * LANES:(g + 1) * LANES]
            dg = (x2b + mg) + e2g
            better = dg < run_min
            run_min = jnp.minimum(run_min, dg)
            run_grp = jnp.where(better, j * GRPS + g, run_grp)
        return run_min, run_grp

    init = (jnp.full((R3, SUB, LANES), jnp.inf, jnp.float32),
            jnp.zeros((R3, SUB, LANES), jnp.int32))
    run_min, run_grp = lax.fori_loop(0, NUM_CODES // C_BLK, chunk, init)

    rm = run_min.reshape(R_BLK, LANES)
    rg = run_grp.reshape(R_BLK, LANES)
    best = jnp.min(rm, axis=1, keepdims=True)
    code = (rg * LANES + lax.broadcasted_iota(
        jnp.int32, (R_BLK, LANES), 1)).astype(jnp.float32)
    idx_f = jnp.min(jnp.where(rm == best, code, jnp.float32(NUM_CODES)),
                    axis=1)
    idx_ref[...] = idx_f.astype(jnp.int32)
    acc_ref[0] += jnp.sum(best)

    @pl.when(i == pl.num_programs(0) - 1)
    def _():
        loss_ref[0, 0] = acc_ref[0]


def _argmin_call(flat, embedding, embedding_t):
    return pl.pallas_call(
        _argmin_body,
        grid=(ROWS // R_BLK,),
        in_specs=[
            pl.BlockSpec((R_BLK, DIM), lambda i: (i, 0)),
            pl.BlockSpec((NUM_CODES, DIM), lambda i: (0, 0)),
            pl.BlockSpec((DIM, NUM_CODES), lambda i: (0, 0)),
        ],
        out_specs=[
            pl.BlockSpec((R_BLK,), lambda i: (i,)),
            pl.BlockSpec(memory_space=pltpu.SMEM),
        ],
        out_shape=[
            jax.ShapeDtypeStruct((ROWS,), jnp.int32),
            jax.ShapeDtypeStruct((1, 1), jnp.float32),
        ],
        scratch_shapes=[pltpu.VMEM((SUB, NUM_CODES), jnp.float32),
                        pltpu.SMEM((1,), jnp.float32)],
    )(flat, embedding, embedding_t)


def _gather_call(embedding, idx_flat):
    mesh = plsc.VectorSubcoreMesh(core_axis_name="c", subcore_axis_name="s")

    @functools.partial(
        pl.kernel,
        mesh=mesh,
        compiler_params=pltpu.CompilerParams(use_tc_tiling_on_sc=False),
        out_type=jax.ShapeDtypeStruct((ROWS, DIM), jnp.float32),
        scratch_types=[
            pltpu.VMEM((ROWS_PER_WORKER,), jnp.int32),
            pltpu.VMEM((ROWS_PER_WORKER, DIM), jnp.float32),
            pltpu.SemaphoreType.DMA,
        ],
    )
    def gather(table_hbm, idx_hbm, out_hbm, idx_v, rows_v, sem):
        wid = lax.axis_index("s") * SC_CORES + lax.axis_index("c")
        base = wid * ROWS_PER_WORKER
        pltpu.sync_copy(idx_hbm.at[pl.ds(base, ROWS_PER_WORKER)], idx_v)
        pltpu.async_copy(table_hbm.at[idx_v], rows_v, sem).wait()
        pltpu.sync_copy(rows_v, out_hbm.at[pl.ds(base, ROWS_PER_WORKER)])

    return gather(embedding, idx_flat)


def kernel(z_e, embedding):
    B, D, H, W = z_e.shape
    flat = jnp.transpose(z_e, (0, 2, 3, 1)).reshape(-1, D)
    idx_flat, loss_acc = _argmin_call(flat, embedding, embedding.T)
    z_q_flat = _gather_call(embedding, idx_flat)
    z_q = jnp.transpose(z_q_flat.reshape(B, H, W, D), (0, 3, 1, 2))
    z_q_st = z_e + lax.stop_gradient(z_q - z_e)
    loss = loss_acc[0, 0] / (B * H * W * D)
    return (z_q_st, loss, idx_flat.reshape(B, H, W))
